# baseline (speedup 1.0000x reference)
import dataclasses
import functools

import jax
import jax.numpy as jnp
from jax import lax
from jax.experimental import pallas as pl
from jax.experimental.pallas import tpu as pltpu
from jax.experimental.pallas import tpu_sc as plsc

BATCH = 16384
EMBED_DIM = 64
PACKED = 2 * EMBED_DIM
N_USERS = 1000000
N_PRODUCTS = 100000

NC = 2
NS = 16
NW = NC * NS
BPW = BATCH // NW
CHUNK = 128
CPW = BPW // CHUNK

_sc_mesh = plsc.VectorSubcoreMesh(core_axis_name="c", subcore_axis_name="s")

_sc_params = pltpu.CompilerParams()
if "needs_layout_passes" in pltpu.CompilerParams.__dataclass_fields__:
    _sc_params = dataclasses.replace(_sc_params, needs_layout_passes=False)


HU = N_USERS // 2
HP = N_PRODUCTS // 2
UPW = (HU // NW) // 8 * 8
UREM = HU - UPW * NW
UCH = 128
PWN = 25
PPW = HP // PWN
PCH = 80
RCH = 128


def _sc_repack(user_table, product_table):
    @functools.partial(
        pl.kernel,
        mesh=_sc_mesh,
        compiler_params=_sc_params,
        out_type=(
            jax.ShapeDtypeStruct((HU, PACKED), jnp.float32),
            jax.ShapeDtypeStruct((HP, PACKED), jnp.float32),
        ),
        scratch_types=[
            pltpu.VMEM((RCH, EMBED_DIM), jnp.float32),
            pltpu.VMEM((RCH, EMBED_DIM), jnp.float32),
            pltpu.VMEM((RCH, PACKED), jnp.float32),
            pltpu.SemaphoreType.DMA,
        ],
    )
    def k(ut_hbm, pt_hbm, pu_hbm, pp_hbm, a_buf, b_buf, obuf, sem):
        wid = lax.axis_index("s") * NC + lax.axis_index("c")

        def pack_span(src, dst, q0, nrows):
            ca = pltpu.async_copy(
                src.at[pl.ds(q0, nrows)], a_buf.at[pl.ds(0, nrows)], sem)
            cb = pltpu.async_copy(
                src.at[pl.ds(src.shape[0] // 2 + q0, nrows)],
                b_buf.at[pl.ds(0, nrows)], sem)
            ca.wait()
            cb.wait()

            @pl.loop(0, nrows)
            def _(r):
                for l in range(EMBED_DIM // 16):
                    sl = pl.ds(16 * l, 16)
                    sl2 = pl.ds(EMBED_DIM + 16 * l, 16)
                    obuf.at[r, sl][...] = a_buf.at[r, sl][...]
                    obuf.at[r, sl2][...] = b_buf.at[r, sl][...]

            pltpu.sync_copy(obuf.at[pl.ds(0, nrows)], dst.at[pl.ds(q0, nrows)])

        @pl.loop(0, UPW // UCH)
        def _(c):
            pack_span(ut_hbm, pu_hbm, wid * UPW + c * UCH, UCH)

        @pl.when(wid == 0)
        def _():
            @pl.loop(0, UREM // 96)
            def _(c):
                pack_span(ut_hbm, pu_hbm, UPW * NW + c * 96, 96)

        @pl.when(wid < PWN)
        def _():
            @pl.loop(0, PPW // PCH)
            def _(c):
                pack_span(pt_hbm, pp_hbm, wid * PPW + c * PCH, PCH)

    return k(user_table, product_table)


def _repack_body(a, b, out):
    out[...] = jnp.concatenate([a[...], b[...]], axis=1)


def _tc_repack(table, blk):
    nmaj = table.shape[0] // 2
    nblk = nmaj // blk
    return pl.pallas_call(
        _repack_body,
        grid=(nblk,),
        in_specs=[pl.BlockSpec((blk, EMBED_DIM), lambda i: (i, 0)),
                  pl.BlockSpec((blk, EMBED_DIM),
                               lambda i, _n=nblk: (i + _n, 0))],
        out_specs=pl.BlockSpec((blk, PACKED), lambda i: (i, 0)),
        out_shape=jax.ShapeDtypeStruct((nmaj, PACKED), jnp.float32),
    )(table, table)


def _sc_gather(packed, qidx):
    @functools.partial(
        pl.kernel,
        mesh=_sc_mesh,
        compiler_params=_sc_params,
        out_type=jax.ShapeDtypeStruct((BATCH, PACKED), jnp.float32),
        scratch_types=[
            pltpu.VMEM((BPW,), jnp.int32),
            pltpu.VMEM((CHUNK, PACKED), jnp.float32),
            pltpu.SemaphoreType.DMA,
        ],
    )
    def k(t_hbm, q_hbm, o_hbm, q_v, r_v, sem):
        wid = lax.axis_index("s") * NC + lax.axis_index("c")
        base = wid * BPW
        pltpu.sync_copy(q_hbm.at[pl.ds(base, BPW)], q_v)
        for c in range(CPW):
            pltpu.async_copy(
                t_hbm.at[q_v.at[pl.ds(c * CHUNK, CHUNK)]], r_v, sem).wait()
            pltpu.sync_copy(r_v, o_hbm.at[pl.ds(base + c * CHUNK, CHUNK)])

    return k(packed, qidx)


def _mlp_body(bu, bp, glue, w1, b1, w2, b2, w3a, w3b, w3c, b3, w4, b4, out):
    g = glue[...]
    bu_ = bu[...]
    bp_ = bp[...]
    ue = jnp.where(g[:, 0:1] > 0, bu_[:, EMBED_DIM:], bu_[:, :EMBED_DIM])
    pe = jnp.where(g[:, 1:2] > 0, bp_[:, EMBED_DIM:], bp_[:, :EMBED_DIM])
    m = ue * pe
    uf = g[:, 2:13]
    bd = g[:, 13:16]
    ufeat = jnp.maximum(
        jnp.dot(uf, w1[...], preferred_element_type=jnp.float32) + b1[...], 0.0)
    bfeat = jnp.maximum(
        jnp.dot(bd, w2[...], preferred_element_type=jnp.float32) + b2[...], 0.0)
    h = (jnp.dot(m, w3a[...], preferred_element_type=jnp.float32)
         + jnp.dot(ufeat, w3b[...], preferred_element_type=jnp.float32)
         + jnp.dot(bfeat, w3c[...], preferred_element_type=jnp.float32)
         + b3[...])
    h = jnp.maximum(h, 0.0)
    logit = jnp.dot(h, w4[...], preferred_element_type=jnp.float32) + b4[...]
    out[...] = jax.nn.sigmoid(logit)


_TC_BLOCK = 2048


def _tc_mlp(bu, bp, glue, w1, b1, w2, b2, w3a, w3b, w3c, b3, w4, b4):
    def row_block(width):
        return pl.BlockSpec((_TC_BLOCK, width), lambda i: (i, 0))

    def whole(a):
        return pl.BlockSpec(a.shape, lambda i: (0, 0))

    return pl.pallas_call(
        _mlp_body,
        grid=(BATCH // _TC_BLOCK,),
        in_specs=[row_block(PACKED), row_block(PACKED), row_block(16),
                  whole(w1), whole(b1), whole(w2), whole(b2),
                  whole(w3a), whole(w3b), whole(w3c), whole(b3),
                  whole(w4), whole(b4)],
        out_specs=row_block(1),
        out_shape=jax.ShapeDtypeStruct((BATCH, 1), jnp.float32),
    )(bu, bp, glue, w1, b1, w2, b2, w3a, w3b, w3c, b3, w4, b4)


@jax.jit
def _run(user_ids, product_ids, user_features, behavior_data,
         user_table, product_table, W1, b1, W2, b2, W3, b3, W4, b4):
    hu = N_USERS // 2
    hp = N_PRODUCTS // 2
    uq = jnp.where(user_ids >= hu, user_ids - hu, user_ids)
    pq = jnp.where(product_ids >= hp, product_ids - hp, product_ids)
    su = (user_ids >= hu).astype(jnp.float32).reshape(BATCH, 1)
    sp = (product_ids >= hp).astype(jnp.float32).reshape(BATCH, 1)
    glue = jnp.concatenate([su, sp, user_features, behavior_data], axis=1)

    packed_u, packed_p = _sc_repack(user_table, product_table)
    bp = _sc_gather(packed_p, pq)
    bu = _sc_gather(packed_u, uq)

    return _tc_mlp(
        bu, bp, glue,
        W1.T, b1.reshape(1, 32), W2.T, b2.reshape(1, 32),
        W3[:, :EMBED_DIM].T, W3[:, EMBED_DIM:EMBED_DIM + 32].T,
        W3[:, EMBED_DIM + 32:].T, b3.reshape(1, 32),
        W4.T, b4.reshape(1, 1))


def kernel(user_ids, product_ids, user_features, behavior_data,
           user_table, product_table, W1, b1, W2, b2, W3, b3, W4, b4):
    return _run(user_ids, product_ids, user_features, behavior_data,
                user_table, product_table, W1, b1, W2, b2, W3, b3, W4, b4)


# speedup vs baseline: 1.2983x; 1.2983x over previous
import dataclasses
import functools

import jax
import jax.numpy as jnp
from jax import lax
from jax.experimental import pallas as pl
from jax.experimental.pallas import tpu as pltpu
from jax.experimental.pallas import tpu_sc as plsc

BATCH = 16384
EMBED_DIM = 64
PACKED = 2 * EMBED_DIM
N_USERS = 1000000
N_PRODUCTS = 100000

NC = 2
NS = 16
NW = NC * NS
BPW = BATCH // NW
CHUNK = 128
CPW = BPW // CHUNK

_sc_mesh = plsc.VectorSubcoreMesh(core_axis_name="c", subcore_axis_name="s")

_sc_params = pltpu.CompilerParams()
if "needs_layout_passes" in pltpu.CompilerParams.__dataclass_fields__:
    _sc_params = dataclasses.replace(_sc_params, needs_layout_passes=False)


def _repack_body(a, b, out):
    out[...] = jnp.concatenate([a[...], b[...]], axis=1)


def _tc_repack(table, blk):
    nmaj = table.shape[0] // 2
    nblk = nmaj // blk
    return pl.pallas_call(
        _repack_body,
        grid=(nblk,),
        in_specs=[pl.BlockSpec((blk, EMBED_DIM), lambda i: (i, 0)),
                  pl.BlockSpec((blk, EMBED_DIM),
                               lambda i, _n=nblk: (i + _n, 0))],
        out_specs=pl.BlockSpec((blk, PACKED), lambda i: (i, 0)),
        out_shape=jax.ShapeDtypeStruct((nmaj, PACKED), jnp.float32),
    )(table, table)


def _sc_gather(packed, qidx):
    @functools.partial(
        pl.kernel,
        mesh=_sc_mesh,
        compiler_params=_sc_params,
        out_type=jax.ShapeDtypeStruct((BATCH, PACKED), jnp.float32),
        scratch_types=[
            pltpu.VMEM((BPW,), jnp.int32),
            pltpu.VMEM((CHUNK, PACKED), jnp.float32),
            pltpu.SemaphoreType.DMA,
        ],
    )
    def k(t_hbm, q_hbm, o_hbm, q_v, r_v, sem):
        wid = lax.axis_index("s") * NC + lax.axis_index("c")
        base = wid * BPW
        pltpu.sync_copy(q_hbm.at[pl.ds(base, BPW)], q_v)
        for c in range(CPW):
            pltpu.async_copy(
                t_hbm.at[q_v.at[pl.ds(c * CHUNK, CHUNK)]], r_v, sem).wait()
            pltpu.sync_copy(r_v, o_hbm.at[pl.ds(base + c * CHUNK, CHUNK)])

    return k(packed, qidx)


def _mlp_body(bu, bp, glue, w1, b1, w2, b2, w3a, w3b, w3c, b3, w4, b4, out):
    g = glue[...]
    bu_ = bu[...]
    bp_ = bp[...]
    ue = jnp.where(g[:, 0:1] > 0, bu_[:, EMBED_DIM:], bu_[:, :EMBED_DIM])
    pe = jnp.where(g[:, 1:2] > 0, bp_[:, EMBED_DIM:], bp_[:, :EMBED_DIM])
    m = ue * pe
    uf = g[:, 2:13]
    bd = g[:, 13:16]
    ufeat = jnp.maximum(
        jnp.dot(uf, w1[...], preferred_element_type=jnp.float32) + b1[...], 0.0)
    bfeat = jnp.maximum(
        jnp.dot(bd, w2[...], preferred_element_type=jnp.float32) + b2[...], 0.0)
    h = (jnp.dot(m, w3a[...], preferred_element_type=jnp.float32)
         + jnp.dot(ufeat, w3b[...], preferred_element_type=jnp.float32)
         + jnp.dot(bfeat, w3c[...], preferred_element_type=jnp.float32)
         + b3[...])
    h = jnp.maximum(h, 0.0)
    logit = jnp.dot(h, w4[...], preferred_element_type=jnp.float32) + b4[...]
    out[...] = jax.nn.sigmoid(logit)


_TC_BLOCK = 2048


def _tc_mlp(bu, bp, glue, w1, b1, w2, b2, w3a, w3b, w3c, b3, w4, b4):
    def row_block(width):
        return pl.BlockSpec((_TC_BLOCK, width), lambda i: (i, 0))

    def whole(a):
        return pl.BlockSpec(a.shape, lambda i: (0, 0))

    return pl.pallas_call(
        _mlp_body,
        grid=(BATCH // _TC_BLOCK,),
        in_specs=[row_block(PACKED), row_block(PACKED), row_block(16),
                  whole(w1), whole(b1), whole(w2), whole(b2),
                  whole(w3a), whole(w3b), whole(w3c), whole(b3),
                  whole(w4), whole(b4)],
        out_specs=row_block(1),
        out_shape=jax.ShapeDtypeStruct((BATCH, 1), jnp.float32),
    )(bu, bp, glue, w1, b1, w2, b2, w3a, w3b, w3c, b3, w4, b4)


@jax.jit
def _run(user_ids, product_ids, user_features, behavior_data,
         user_table, product_table, W1, b1, W2, b2, W3, b3, W4, b4):
    hu = N_USERS // 2
    hp = N_PRODUCTS // 2
    uq = jnp.where(user_ids >= hu, user_ids - hu, user_ids)
    pq = jnp.where(product_ids >= hp, product_ids - hp, product_ids)
    su = (user_ids >= hu).astype(jnp.float32).reshape(BATCH, 1)
    sp = (product_ids >= hp).astype(jnp.float32).reshape(BATCH, 1)
    glue = jnp.concatenate([su, sp, user_features, behavior_data], axis=1)

    packed_p = _tc_repack(product_table, 5000)
    bp = _sc_gather(packed_p, pq)
    packed_u = _tc_repack(user_table, 5000)
    bu = _sc_gather(packed_u, uq)

    return _tc_mlp(
        bu, bp, glue,
        W1.T, b1.reshape(1, 32), W2.T, b2.reshape(1, 32),
        W3[:, :EMBED_DIM].T, W3[:, EMBED_DIM:EMBED_DIM + 32].T,
        W3[:, EMBED_DIM + 32:].T, b3.reshape(1, 32),
        W4.T, b4.reshape(1, 1))


def kernel(user_ids, product_ids, user_features, behavior_data,
           user_table, product_table, W1, b1, W2, b2, W3, b3, W4, b4):
    return _run(user_ids, product_ids, user_features, behavior_data,
                user_table, product_table, W1, b1, W2, b2, W3, b3, W4, b4)
